# baseline (speedup 1.0000x reference)
import functools

import jax
import jax.numpy as jnp
from jax import lax
from jax.experimental import pallas as pl
from jax.experimental.pallas import tpu as pltpu
from jax.experimental.pallas import tpu_sc as plsc

N = 10000
E = 640000
H = 0.1
_INTERPRET = False


def _stats_matmul(xs, krons, rows_per_blk, offsets=None, rows=None):
    R = rows if rows is not None else xs[0].shape[0]
    if offsets is None:
        offsets = [0] * len(xs)
    Lout = krons[0].shape[1]
    nb = R // rows_per_blk
    assert R % rows_per_blk == 0

    def body(*refs):
        bi = pl.program_id(0)
        nx = len(xs)
        x_refs = refs[:nx]
        k_refs = refs[nx:2 * nx]
        h_ref, st_ref = refs[2 * nx], refs[2 * nx + 1]
        h = jnp.zeros((rows_per_blk, Lout), jnp.float32)
        for xr, kr in zip(x_refs, k_refs):
            h = h + jnp.dot(xr[...], kr[...], preferred_element_type=jnp.float32)
        h_ref[...] = h
        ps = jnp.sum(h, axis=0, keepdims=True)
        ps2 = jnp.sum(h * h, axis=0, keepdims=True)
        if Lout > 128:
            ps = ps.reshape(Lout // 128, 128).sum(axis=0, keepdims=True)
            ps2 = ps2.reshape(Lout // 128, 128).sum(axis=0, keepdims=True)

        @pl.when(bi == 0)
        def _():
            st_ref[...] = jnp.zeros((2, 128), jnp.float32)

        st_ref[0:1, :] += ps
        st_ref[1:2, :] += ps2

    in_specs = (
        [pl.BlockSpec((rows_per_blk, 128), functools.partial(lambda o, b: (b + o, 0), o))
         for o in offsets]
        + [pl.BlockSpec((128, Lout), lambda b: (0, 0)) for _ in krons]
    )
    out_specs = [
        pl.BlockSpec((rows_per_blk, Lout), lambda b: (b, 0)),
        pl.BlockSpec((2, 128), lambda b: (0, 0)),
    ]
    h, st = pl.pallas_call(
        body,
        grid=(nb,),
        in_specs=in_specs,
        out_specs=out_specs,
        out_shape=[
            jax.ShapeDtypeStruct((R, Lout), jnp.float32),
            jax.ShapeDtypeStruct((2, 128), jnp.float32),
        ],
        interpret=_INTERPRET,
    )(*xs, *krons)
    return h, st


def _apply_matmul(h, stats, kron2, count, rows_per_blk, resid=None, hscale=None):
    R, Lin = h.shape
    Lout = kron2.shape[1]
    nb = R // rows_per_blk
    assert R % rows_per_blk == 0

    def body(*refs):
        if resid is not None:
            h_ref, st_ref, k_ref, r_ref, o_ref = refs
        else:
            h_ref, st_ref, k_ref, o_ref = refs
            r_ref = None
        s = jnp.sum(st_ref[0, :])
        s2 = jnp.sum(st_ref[1, :])
        mean = s / count
        var = s2 / count - mean * mean
        inv = lax.rsqrt(var + 1e-5)
        g = jnp.tanh((h_ref[...] - mean) * inv)
        d = jnp.dot(g, k_ref[...], preferred_element_type=jnp.float32)
        if r_ref is not None:
            o_ref[...] = r_ref[...] + hscale * d
        else:
            o_ref[...] = d

    ins = [h, stats, kron2] + ([resid] if resid is not None else [])
    in_specs = [
        pl.BlockSpec((rows_per_blk, Lin), lambda b: (b, 0)),
        pl.BlockSpec((2, 128), lambda b: (0, 0)),
        pl.BlockSpec((Lin, Lout), lambda b: (0, 0)),
    ] + ([pl.BlockSpec((rows_per_blk, Lout), lambda b: (b, 0))] if resid is not None else [])
    out = pl.pallas_call(
        body,
        grid=(nb,),
        in_specs=in_specs,
        out_specs=pl.BlockSpec((rows_per_blk, Lout), lambda b: (b, 0)),
        out_shape=jax.ShapeDtypeStruct((R, Lout), jnp.float32),
        interpret=_INTERPRET,
    )(*ins)
    return out


def _open_stats(x_b3m, w1, blk_m):
    M = x_b3m.shape[2]
    nb = M // blk_m
    assert M % blk_m == 0

    def body(x_ref, w_ref, h_ref, st_ref, acc_ref):
        bi = pl.program_id(0)
        h = lax.dot_general(w_ref[...], x_ref[0],
                            (((1,), (0,)), ((), ())),
                            preferred_element_type=jnp.float32)
        h_ref[...] = h

        @pl.when(bi == 0)
        def _():
            acc_ref[0] = 0.0
            acc_ref[1] = 0.0

        acc_ref[0] += jnp.sum(h)
        acc_ref[1] += jnp.sum(h * h)

        @pl.when(bi == nb - 1)
        def _():
            o = jnp.ones((1, 128), jnp.float32)
            st_ref[0:1, :] = o * (acc_ref[0] / 128.0)
            st_ref[1:2, :] = o * (acc_ref[1] / 128.0)

    h, st = pl.pallas_call(
        body,
        grid=(nb,),
        in_specs=[
            pl.BlockSpec((1, 3, blk_m), lambda b: (0, 0, b)),
            pl.BlockSpec((16, 3), lambda b: (0, 0)),
        ],
        out_specs=[
            pl.BlockSpec((16, blk_m), lambda b: (0, b)),
            pl.BlockSpec((2, 128), lambda b: (0, 0)),
        ],
        out_shape=[
            jax.ShapeDtypeStruct((16, M), jnp.float32),
            jax.ShapeDtypeStruct((2, 128), jnp.float32),
        ],
        scratch_shapes=[pltpu.SMEM((2,), jnp.float32)],
        interpret=_INTERPRET,
    )(x_b3m, w1)
    return h, st


def _open_apply(h_cm, stats, w2, count, blk_m):
    M = h_cm.shape[1]
    nb = M // blk_m

    def body(h_ref, st_ref, w_ref, o_ref):
        s = jnp.sum(st_ref[0, :])
        s2 = jnp.sum(st_ref[1, :])
        mean = s / count
        inv = lax.rsqrt(s2 / count - mean * mean + 1e-5)
        g = jnp.tanh((h_ref[...] - mean) * inv)
        o_ref[...] = lax.dot_general(g, w_ref[...], (((0,), (1,)), ((), ())),
                                     preferred_element_type=jnp.float32)

    return pl.pallas_call(
        body,
        grid=(nb,),
        in_specs=[
            pl.BlockSpec((16, blk_m), lambda b: (0, b)),
            pl.BlockSpec((2, 128), lambda b: (0, 0)),
            pl.BlockSpec((16, 16), lambda b: (0, 0)),
        ],
        out_specs=pl.BlockSpec((blk_m, 16), lambda b: (b, 0)),
        out_shape=jax.ShapeDtypeStruct((M, 16), jnp.float32),
        interpret=_INTERPRET,
    )(h_cm, stats, w2)


def _node_double_layer(xs, krons, kron2, count, resid=None, hscale=None):
    Lout = kron2.shape[1]
    R = xs[0].shape[0]

    def body(*refs):
        nx = len(xs)
        x_refs = refs[:nx]
        k_refs = refs[nx:2 * nx]
        k2_ref = refs[2 * nx]
        if resid is not None:
            r_ref, o_ref = refs[2 * nx + 1], refs[2 * nx + 2]
        else:
            r_ref, o_ref = None, refs[2 * nx + 1]
        h = jnp.zeros((R, krons[0].shape[1]), jnp.float32)
        for xr, kr in zip(x_refs, k_refs):
            h = h + jnp.dot(xr[...], kr[...], preferred_element_type=jnp.float32)
        mean = jnp.sum(h) / count
        var = jnp.sum(h * h) / count - mean * mean
        g = jnp.tanh((h - mean) * lax.rsqrt(var + 1e-5))
        d = jnp.dot(g, k2_ref[...], preferred_element_type=jnp.float32)
        if r_ref is not None:
            o_ref[...] = r_ref[...] + hscale * d
        else:
            o_ref[...] = d

    ins = list(xs) + list(krons) + [kron2] + ([resid] if resid is not None else [])
    out = pl.pallas_call(
        body,
        out_shape=jax.ShapeDtypeStruct((R, Lout), jnp.float32),
        interpret=_INTERPRET,
    )(*ins)
    return out


def _close_mlp1(xn_r8, kron_close, kron_lin1, b1t):
    def body(x_ref, kc_ref, k1_ref, b_ref, o_ref):
        y = jnp.dot(x_ref[...], kc_ref[...], preferred_element_type=jnp.float32)
        t = jnp.dot(y, k1_ref[...], preferred_element_type=jnp.float32) + b_ref[...]
        o_ref[...] = jnp.where(t > 0, t, jnp.exp(jnp.minimum(t, 0.0)) - 1.0)

    return pl.pallas_call(
        body,
        out_shape=jax.ShapeDtypeStruct((xn_r8.shape[0], kron_lin1.shape[1]), jnp.float32),
        interpret=_INTERPRET,
    )(xn_r8, kron_close, kron_lin1, b1t)


def _close_mlp2(a, w2t, b2, rows_per_blk):
    R = a.shape[0]
    nb = R // rows_per_blk

    def body(a_ref, w_ref, b_ref, o_ref):
        z = jnp.dot(a_ref[...], w_ref[...], preferred_element_type=jnp.float32) + b_ref[...]
        m = jnp.max(z, axis=1, keepdims=True)
        lse = m + jnp.log(jnp.sum(jnp.exp(z - m), axis=1, keepdims=True))
        o_ref[...] = z - lse

    return pl.pallas_call(
        body,
        grid=(nb,),
        in_specs=[
            pl.BlockSpec((rows_per_blk, 256), lambda b: (b, 0)),
            pl.BlockSpec((256, 1024), lambda b: (0, 0)),
            pl.BlockSpec((1, 1024), lambda b: (0, 0)),
        ],
        out_specs=pl.BlockSpec((rows_per_blk, 1024), lambda b: (b, 0)),
        out_shape=jax.ShapeDtypeStruct((R, 1024), jnp.float32),
        interpret=_INTERPRET,
    )(a, w2t, b2)


_GATHER_WIN = 128
_SC_PARAMS = pltpu.CompilerParams(use_tc_tiling_on_sc=False)


_GGRP = 2048


def _sc_gather(table_rows, idx2e):
    n_idx = idx2e.shape[0]
    n_grp = n_idx // _GGRP
    assert n_idx % _GGRP == 0
    mesh = plsc.VectorSubcoreMesh(core_axis_name="c", subcore_axis_name="s")

    @functools.partial(
        pl.kernel,
        out_type=jax.ShapeDtypeStruct((n_idx, 16), jnp.float32),
        mesh=mesh,
        scratch_types=[
            pltpu.VMEM((_GGRP,), jnp.int32),
            pltpu.VMEM((_GGRP, 16), jnp.float32),
            pltpu.SemaphoreType.DMA,
        ],
        compiler_params=_SC_PARAMS,
    )
    def k(x_hbm, i_hbm, o_hbm, ivm, rows, sem):
        wid = lax.axis_index("c") * 16 + lax.axis_index("s")

        @pl.loop(0, n_grp)
        def _(g):
            @pl.when(g % 32 == wid)
            def _():
                base = g * _GGRP
                pltpu.sync_copy(i_hbm.at[pl.ds(base, _GGRP)], ivm)
                hs = [
                    pltpu.async_copy(
                        x_hbm.at[ivm.at[pl.ds(k * 128, 128)]],
                        rows.at[pl.ds(k * 128, 128)],
                        sem,
                    )
                    for k in range(_GGRP // 128)
                ]
                for h in hs:
                    h.wait()
                pltpu.sync_copy(rows, o_hbm.at[pl.ds(base, _GGRP)])

    return k(table_rows, idx2e)


_SGRP = 1024


def _sc_scatter(xe_rows, edge_index3, zeros_rows):
    mesh = plsc.VectorSubcoreMesh(core_axis_name="c", subcore_axis_name="s")
    n_grp = E // _SGRP
    kk = _SGRP // 128

    @functools.partial(
        pl.kernel,
        out_type=jax.ShapeDtypeStruct((2, 2, N, 16), jnp.float32),
        mesh=mesh,
        scratch_types=[
            pltpu.VMEM_SHARED((N, 16), jnp.float32),
            pltpu.VMEM_SHARED((N, 16), jnp.float32),
            pltpu.VMEM((kk, 128), jnp.int32),
            pltpu.VMEM((kk, 128), jnp.int32),
            pltpu.VMEM((_SGRP, 16), jnp.float32),
            pltpu.SemaphoreType.DMA,
            pltpu.SemaphoreType.DMA,
        ],
        compiler_params=_SC_PARAMS,
    )
    def k(xe_hbm, ij_hbm, z_hbm, o_hbm, acc_i, acc_j, ii, jj, rows, sem_l, sem_s):
        c = lax.axis_index("c")
        s = lax.axis_index("s")
        wid = c * 16 + s

        @pl.when(s == 0)
        def _():
            pltpu.sync_copy(z_hbm, acc_i)
            pltpu.sync_copy(z_hbm, acc_j)

        plsc.subcore_barrier()

        @pl.loop(0, n_grp)
        def _(g):
            @pl.when(g % 32 == wid)
            def _():
                h0 = pltpu.async_copy(xe_hbm.at[pl.ds(g * _SGRP, _SGRP)], rows,
                                      sem_l)
                pltpu.sync_copy(ij_hbm.at[0, pl.ds(g * kk, kk)], ii)
                pltpu.sync_copy(ij_hbm.at[1, pl.ds(g * kk, kk)], jj)
                h0.wait()
                hs = [
                    pltpu.async_copy(rows.at[pl.ds(k * 128, 128)],
                                     acc_i.at[ii.at[k]], sem_s, add=True)
                    for k in range(kk)
                ] + [
                    pltpu.async_copy(rows.at[pl.ds(k * 128, 128)],
                                     acc_j.at[jj.at[k]], sem_s, add=True)
                    for k in range(kk)
                ]
                for h in hs:
                    h.wait()

        plsc.subcore_barrier()

        @pl.when(s == 0)
        def _():
            pltpu.sync_copy(acc_i, o_hbm.at[c, 0])
            pltpu.sync_copy(acc_j, o_hbm.at[c, 1])

    return k(xe_rows, edge_index3, zeros_rows)


def _kron8(w):
    return jnp.kron(jnp.eye(8, dtype=jnp.float32), w.T)


def kernel(xn, xe, edge_index, K1Nopen, K2Nopen, K1Eopen, K2Eopen, KNclose,
           alpha, KE1, KE2, KN1, KN2, lin1_w, lin1_b, lin2_w, lin2_b):
    f32 = jnp.float32
    idx2e = edge_index.reshape(2 * E)

    kA, kB, kC, k2e = [], [], [], []
    kU, kV, kR, k2n = [], [], [], []
    for i in range(KE1.shape[0]):
        P, C, G = KE1[i][:, 0:16], KE1[i][:, 16:32], KE1[i][:, 32:48]
        kA.append(_kron8(P / 2 + G))
        kB.append(_kron8(P / 2 - G))
        kC.append(_kron8(C))
        k2e.append(_kron8(KE2[i]))
        Pn, Qn, Rn = KN1[i][:, 0:16], KN1[i][:, 16:32], KN1[i][:, 32:48]
        kU.append(_kron8(Pn / 2 + Qn))
        kV.append(_kron8(Pn / 2 - Qn))
        kR.append(_kron8(Rn))
        k2n.append(_kron8(KN2[i]))
    kron_close = _kron8(KNclose)
    kron_lin1 = jnp.kron(jnp.eye(8, dtype=f32), lin1_w.T)
    b1t = jnp.tile(lin1_b, 8).reshape(1, 2048)
    w2t = lin2_w.T
    b2 = lin2_b.reshape(1, 1024)
    zeros_rows = jnp.zeros((N, 16), f32)

    hn, stn = _open_stats(xn, K1Nopen, blk_m=N)
    xn_r8 = _open_apply(hn, stn, K2Nopen, float(16 * N), blk_m=N).reshape(N // 8, 128)

    he, ste = _open_stats(xe, K1Eopen, blk_m=6400)
    xe_r8 = _open_apply(he, ste, K2Eopen, float(16 * E), blk_m=6400).reshape(E // 8, 128)

    for i in range(KE1.shape[0]):
        xij = _sc_gather(xn_r8.reshape(N, 16), idx2e).reshape(2 * E // 8, 128)
        h, st = _stats_matmul([xij, xij, xe_r8], [kA[i], kB[i], kC[i]],
                              rows_per_blk=8000,
                              offsets=[0, (E // 8) // 8000, 0], rows=E // 8)
        xe_r8 = _apply_matmul(h, st, k2e[i], float(16 * E), rows_per_blk=8000,
                              resid=xe_r8, hscale=H)
        P = _sc_scatter(xe_r8.reshape(E, 16),
                        edge_index.reshape(2, E // 128, 128), zeros_rows)
        Pr = P.reshape(4, N // 8, 128)
        si = Pr[0] + Pr[2]
        sj = Pr[1] + Pr[3]
        xn_r8 = _node_double_layer([si, sj, xn_r8], [kU[i], kV[i], kR[i]],
                                   k2n[i], float(16 * N), resid=xn_r8, hscale=H)

    a = _close_mlp1(xn_r8, kron_close, kron_lin1, b1t)
    out = _close_mlp2(a.reshape(N, 256), w2t, b2, rows_per_blk=1000)
    return (out, jax.nn.sigmoid(alpha))


# speedup vs baseline: 1.0350x; 1.0350x over previous
import functools

import jax
import jax.numpy as jnp
from jax import lax
from jax.experimental import pallas as pl
from jax.experimental.pallas import tpu as pltpu
from jax.experimental.pallas import tpu_sc as plsc

N = 10000
E = 640000
H = 0.1
_INTERPRET = False


def _stats_matmul(xs, krons, rows_per_blk, offsets=None, rows=None):
    R = rows if rows is not None else xs[0].shape[0]
    if offsets is None:
        offsets = [0] * len(xs)
    Lout = next(k.shape[1] for k in krons if k is not None)
    nb = R // rows_per_blk
    assert R % rows_per_blk == 0

    real_krons = [k for k in krons if k is not None]

    def body(*refs):
        bi = pl.program_id(0)
        nx = len(xs)
        x_refs = refs[:nx]
        k_refs = list(refs[nx:nx + len(real_krons)])
        h_ref, st_ref = refs[nx + len(real_krons)], refs[nx + len(real_krons) + 1]
        h = jnp.zeros((rows_per_blk, Lout), jnp.float32)
        ki = 0
        for xr, kr in zip(x_refs, krons):
            if kr is None:
                h = h + xr[...]
            else:
                h = h + jnp.dot(xr[...], k_refs[ki][...],
                                preferred_element_type=jnp.float32)
                ki += 1
        h_ref[...] = h
        ps = jnp.sum(h, axis=0, keepdims=True)
        ps2 = jnp.sum(h * h, axis=0, keepdims=True)
        if Lout > 128:
            ps = ps.reshape(Lout // 128, 128).sum(axis=0, keepdims=True)
            ps2 = ps2.reshape(Lout // 128, 128).sum(axis=0, keepdims=True)

        @pl.when(bi == 0)
        def _():
            st_ref[...] = jnp.zeros((2, 128), jnp.float32)

        st_ref[0:1, :] += ps
        st_ref[1:2, :] += ps2

    in_specs = (
        [pl.BlockSpec((rows_per_blk, 128), functools.partial(lambda o, b: (b + o, 0), o))
         for o in offsets]
        + [pl.BlockSpec((128, Lout), lambda b: (0, 0)) for _ in real_krons]
    )
    out_specs = [
        pl.BlockSpec((rows_per_blk, Lout), lambda b: (b, 0)),
        pl.BlockSpec((2, 128), lambda b: (0, 0)),
    ]
    h, st = pl.pallas_call(
        body,
        grid=(nb,),
        in_specs=in_specs,
        out_specs=out_specs,
        out_shape=[
            jax.ShapeDtypeStruct((R, Lout), jnp.float32),
            jax.ShapeDtypeStruct((2, 128), jnp.float32),
        ],
        interpret=_INTERPRET,
    )(*xs, *real_krons)
    return h, st


def _apply_matmul(h, stats, kron2, count, rows_per_blk, resid=None, hscale=None):
    R, Lin = h.shape
    Lout = kron2.shape[1]
    nb = R // rows_per_blk
    assert R % rows_per_blk == 0

    def body(*refs):
        if resid is not None:
            h_ref, st_ref, k_ref, r_ref, o_ref = refs
        else:
            h_ref, st_ref, k_ref, o_ref = refs
            r_ref = None
        s = jnp.sum(st_ref[0, :])
        s2 = jnp.sum(st_ref[1, :])
        mean = s / count
        var = s2 / count - mean * mean
        inv = lax.rsqrt(var + 1e-5)
        g = jnp.tanh((h_ref[...] - mean) * inv)
        d = jnp.dot(g, k_ref[...], preferred_element_type=jnp.float32)
        if r_ref is not None:
            o_ref[...] = r_ref[...] + hscale * d
        else:
            o_ref[...] = d

    ins = [h, stats, kron2] + ([resid] if resid is not None else [])
    in_specs = [
        pl.BlockSpec((rows_per_blk, Lin), lambda b: (b, 0)),
        pl.BlockSpec((2, 128), lambda b: (0, 0)),
        pl.BlockSpec((Lin, Lout), lambda b: (0, 0)),
    ] + ([pl.BlockSpec((rows_per_blk, Lout), lambda b: (b, 0))] if resid is not None else [])
    out = pl.pallas_call(
        body,
        grid=(nb,),
        in_specs=in_specs,
        out_specs=pl.BlockSpec((rows_per_blk, Lout), lambda b: (b, 0)),
        out_shape=jax.ShapeDtypeStruct((R, Lout), jnp.float32),
        interpret=_INTERPRET,
    )(*ins)
    return out


def _open_stats(x_b3m, w1, blk_m):
    M = x_b3m.shape[2]
    nb = M // blk_m
    assert M % blk_m == 0

    def body(x_ref, w_ref, h_ref, st_ref, acc_ref):
        bi = pl.program_id(0)
        h = lax.dot_general(w_ref[...], x_ref[0],
                            (((1,), (0,)), ((), ())),
                            preferred_element_type=jnp.float32)
        h_ref[...] = h

        @pl.when(bi == 0)
        def _():
            acc_ref[0] = 0.0
            acc_ref[1] = 0.0

        acc_ref[0] += jnp.sum(h)
        acc_ref[1] += jnp.sum(h * h)

        @pl.when(bi == nb - 1)
        def _():
            o = jnp.ones((1, 128), jnp.float32)
            st_ref[0:1, :] = o * (acc_ref[0] / 128.0)
            st_ref[1:2, :] = o * (acc_ref[1] / 128.0)

    h, st = pl.pallas_call(
        body,
        grid=(nb,),
        in_specs=[
            pl.BlockSpec((1, 3, blk_m), lambda b: (0, 0, b)),
            pl.BlockSpec((16, 3), lambda b: (0, 0)),
        ],
        out_specs=[
            pl.BlockSpec((16, blk_m), lambda b: (0, b)),
            pl.BlockSpec((2, 128), lambda b: (0, 0)),
        ],
        out_shape=[
            jax.ShapeDtypeStruct((16, M), jnp.float32),
            jax.ShapeDtypeStruct((2, 128), jnp.float32),
        ],
        scratch_shapes=[pltpu.SMEM((2,), jnp.float32)],
        interpret=_INTERPRET,
    )(x_b3m, w1)
    return h, st


def _open_apply(h_cm, stats, w2, count, blk_m):
    M = h_cm.shape[1]
    nb = M // blk_m

    def body(h_ref, st_ref, w_ref, o_ref):
        s = jnp.sum(st_ref[0, :])
        s2 = jnp.sum(st_ref[1, :])
        mean = s / count
        inv = lax.rsqrt(s2 / count - mean * mean + 1e-5)
        g = jnp.tanh((h_ref[...] - mean) * inv)
        o_ref[...] = lax.dot_general(g, w_ref[...], (((0,), (1,)), ((), ())),
                                     preferred_element_type=jnp.float32)

    return pl.pallas_call(
        body,
        grid=(nb,),
        in_specs=[
            pl.BlockSpec((16, blk_m), lambda b: (0, b)),
            pl.BlockSpec((2, 128), lambda b: (0, 0)),
            pl.BlockSpec((16, 16), lambda b: (0, 0)),
        ],
        out_specs=pl.BlockSpec((blk_m, 16), lambda b: (b, 0)),
        out_shape=jax.ShapeDtypeStruct((M, 16), jnp.float32),
        interpret=_INTERPRET,
    )(h_cm, stats, w2)


def _node_double_layer(xs, krons, kron2, count, resid=None, hscale=None):
    Lout = kron2.shape[1]
    R = xs[0].shape[0]

    def body(*refs):
        nx = len(xs)
        x_refs = refs[:nx]
        k_refs = refs[nx:2 * nx]
        k2_ref = refs[2 * nx]
        if resid is not None:
            r_ref, o_ref = refs[2 * nx + 1], refs[2 * nx + 2]
        else:
            r_ref, o_ref = None, refs[2 * nx + 1]
        h = jnp.zeros((R, krons[0].shape[1]), jnp.float32)
        for xr, kr in zip(x_refs, k_refs):
            h = h + jnp.dot(xr[...], kr[...], preferred_element_type=jnp.float32)
        mean = jnp.sum(h) / count
        var = jnp.sum(h * h) / count - mean * mean
        g = jnp.tanh((h - mean) * lax.rsqrt(var + 1e-5))
        d = jnp.dot(g, k2_ref[...], preferred_element_type=jnp.float32)
        if r_ref is not None:
            o_ref[...] = r_ref[...] + hscale * d
        else:
            o_ref[...] = d

    ins = list(xs) + list(krons) + [kron2] + ([resid] if resid is not None else [])
    out = pl.pallas_call(
        body,
        out_shape=jax.ShapeDtypeStruct((R, Lout), jnp.float32),
        interpret=_INTERPRET,
    )(*ins)
    return out


def _edge_tables(xn_r8, ka, kb):
    def body(x_ref, ka_ref, kb_ref, a_ref, b_ref):
        a_ref[...] = jnp.dot(x_ref[...], ka_ref[...],
                             preferred_element_type=jnp.float32)
        b_ref[...] = jnp.dot(x_ref[...], kb_ref[...],
                             preferred_element_type=jnp.float32)

    sh = jax.ShapeDtypeStruct(xn_r8.shape, jnp.float32)
    return pl.pallas_call(
        body, out_shape=[sh, sh], interpret=_INTERPRET,
    )(xn_r8, ka, kb)


def _close_mlp1(xn_r8, kron_close, kron_lin1, b1t):
    def body(x_ref, kc_ref, k1_ref, b_ref, o_ref):
        y = jnp.dot(x_ref[...], kc_ref[...], preferred_element_type=jnp.float32)
        t = jnp.dot(y, k1_ref[...], preferred_element_type=jnp.float32) + b_ref[...]
        o_ref[...] = jnp.where(t > 0, t, jnp.exp(jnp.minimum(t, 0.0)) - 1.0)

    return pl.pallas_call(
        body,
        out_shape=jax.ShapeDtypeStruct((xn_r8.shape[0], kron_lin1.shape[1]), jnp.float32),
        interpret=_INTERPRET,
    )(xn_r8, kron_close, kron_lin1, b1t)


def _close_mlp2(a, w2t, b2, rows_per_blk):
    R = a.shape[0]
    nb = R // rows_per_blk

    def body(a_ref, w_ref, b_ref, o_ref):
        z = jnp.dot(a_ref[...], w_ref[...], preferred_element_type=jnp.float32) + b_ref[...]
        m = jnp.max(z, axis=1, keepdims=True)
        lse = m + jnp.log(jnp.sum(jnp.exp(z - m), axis=1, keepdims=True))
        o_ref[...] = z - lse

    return pl.pallas_call(
        body,
        grid=(nb,),
        in_specs=[
            pl.BlockSpec((rows_per_blk, 256), lambda b: (b, 0)),
            pl.BlockSpec((256, 1024), lambda b: (0, 0)),
            pl.BlockSpec((1, 1024), lambda b: (0, 0)),
        ],
        out_specs=pl.BlockSpec((rows_per_blk, 1024), lambda b: (b, 0)),
        out_shape=jax.ShapeDtypeStruct((R, 1024), jnp.float32),
        interpret=_INTERPRET,
    )(a, w2t, b2)


_GATHER_WIN = 128
_SC_PARAMS = pltpu.CompilerParams(use_tc_tiling_on_sc=False)


_GGRP = 1280


def _sc_gather_add(tab_a, tab_b, edge_index):
    n_grp = E // _GGRP
    assert E % _GGRP == 0
    mesh = plsc.VectorSubcoreMesh(core_axis_name="c", subcore_axis_name="s")

    @functools.partial(
        pl.kernel,
        out_type=jax.ShapeDtypeStruct((E, 16), jnp.float32),
        mesh=mesh,
        scratch_types=[
            pltpu.VMEM((_GGRP,), jnp.int32),
            pltpu.VMEM((_GGRP,), jnp.int32),
            pltpu.VMEM((_GGRP, 16), jnp.float32),
            pltpu.SemaphoreType.DMA,
        ],
        compiler_params=_SC_PARAMS,
    )
    def k(a_hbm, b_hbm, ij_hbm, o_hbm, ivm, jvm, rows, sem):
        wid = lax.axis_index("c") * 16 + lax.axis_index("s")
        nk = _GGRP // 128

        @pl.loop(0, n_grp)
        def _(g):
            @pl.when(g % 32 == wid)
            def _():
                base = g * _GGRP
                pltpu.sync_copy(ij_hbm.at[0, pl.ds(base, _GGRP)], ivm)
                pltpu.sync_copy(ij_hbm.at[1, pl.ds(base, _GGRP)], jvm)
                hs = [
                    pltpu.async_copy(
                        a_hbm.at[ivm.at[pl.ds(k * 128, 128)]],
                        rows.at[pl.ds(k * 128, 128)], sem)
                    for k in range(nk)
                ]
                for h in hs:
                    h.wait()
                hs = [
                    pltpu.async_copy(
                        b_hbm.at[jvm.at[pl.ds(k * 128, 128)]],
                        rows.at[pl.ds(k * 128, 128)], sem, add=True)
                    for k in range(nk)
                ]
                for h in hs:
                    h.wait()
                pltpu.sync_copy(rows, o_hbm.at[pl.ds(base, _GGRP)])

    return k(tab_a, tab_b, edge_index)


_SGRP = 1024


def _sc_scatter(xe_rows, edge_index3, zeros_rows):
    mesh = plsc.VectorSubcoreMesh(core_axis_name="c", subcore_axis_name="s")
    n_grp = E // _SGRP
    kk = _SGRP // 128

    @functools.partial(
        pl.kernel,
        out_type=jax.ShapeDtypeStruct((2, 2, N, 16), jnp.float32),
        mesh=mesh,
        scratch_types=[
            pltpu.VMEM_SHARED((N, 16), jnp.float32),
            pltpu.VMEM_SHARED((N, 16), jnp.float32),
            pltpu.VMEM((kk, 128), jnp.int32),
            pltpu.VMEM((kk, 128), jnp.int32),
            pltpu.VMEM((_SGRP, 16), jnp.float32),
            pltpu.SemaphoreType.DMA,
            pltpu.SemaphoreType.DMA,
        ],
        compiler_params=_SC_PARAMS,
    )
    def k(xe_hbm, ij_hbm, z_hbm, o_hbm, acc_i, acc_j, ii, jj, rows, sem_l, sem_s):
        c = lax.axis_index("c")
        s = lax.axis_index("s")
        wid = c * 16 + s

        @pl.when(s == 0)
        def _():
            pltpu.sync_copy(z_hbm, acc_i)
            pltpu.sync_copy(z_hbm, acc_j)

        plsc.subcore_barrier()

        @pl.loop(0, n_grp)
        def _(g):
            @pl.when(g % 32 == wid)
            def _():
                h0 = pltpu.async_copy(xe_hbm.at[pl.ds(g * _SGRP, _SGRP)], rows,
                                      sem_l)
                pltpu.sync_copy(ij_hbm.at[0, pl.ds(g * kk, kk)], ii)
                pltpu.sync_copy(ij_hbm.at[1, pl.ds(g * kk, kk)], jj)
                h0.wait()
                hs = [
                    pltpu.async_copy(rows.at[pl.ds(k * 128, 128)],
                                     acc_i.at[ii.at[k]], sem_s, add=True)
                    for k in range(kk)
                ] + [
                    pltpu.async_copy(rows.at[pl.ds(k * 128, 128)],
                                     acc_j.at[jj.at[k]], sem_s, add=True)
                    for k in range(kk)
                ]
                for h in hs:
                    h.wait()

        plsc.subcore_barrier()

        @pl.when(s == 0)
        def _():
            pltpu.sync_copy(acc_i, o_hbm.at[c, 0])
            pltpu.sync_copy(acc_j, o_hbm.at[c, 1])

    return k(xe_rows, edge_index3, zeros_rows)


def _kron8(w):
    return jnp.kron(jnp.eye(8, dtype=jnp.float32), w.T)


def kernel(xn, xe, edge_index, K1Nopen, K2Nopen, K1Eopen, K2Eopen, KNclose,
           alpha, KE1, KE2, KN1, KN2, lin1_w, lin1_b, lin2_w, lin2_b):
    f32 = jnp.float32

    kA, kB, kC, k2e = [], [], [], []
    kU, kV, kR, k2n = [], [], [], []
    for i in range(KE1.shape[0]):
        P, C, G = KE1[i][:, 0:16], KE1[i][:, 16:32], KE1[i][:, 32:48]
        kA.append(_kron8(P / 2 + G))
        kB.append(_kron8(P / 2 - G))
        kC.append(_kron8(C))
        k2e.append(_kron8(KE2[i]))
        Pn, Qn, Rn = KN1[i][:, 0:16], KN1[i][:, 16:32], KN1[i][:, 32:48]
        kU.append(_kron8(Pn / 2 + Qn))
        kV.append(_kron8(Pn / 2 - Qn))
        kR.append(_kron8(Rn))
        k2n.append(_kron8(KN2[i]))
    kron_close = _kron8(KNclose)
    kron_lin1 = jnp.kron(jnp.eye(8, dtype=f32), lin1_w.T)
    b1t = jnp.tile(lin1_b, 8).reshape(1, 2048)
    w2t = lin2_w.T
    b2 = lin2_b.reshape(1, 1024)
    zeros_rows = jnp.zeros((N, 16), f32)

    hn, stn = _open_stats(xn, K1Nopen, blk_m=N)
    xn_r8 = _open_apply(hn, stn, K2Nopen, float(16 * N), blk_m=N).reshape(N // 8, 128)

    he, ste = _open_stats(xe, K1Eopen, blk_m=6400)
    xe_r8 = _open_apply(he, ste, K2Eopen, float(16 * E), blk_m=6400).reshape(E // 8, 128)

    for i in range(KE1.shape[0]):
        ya, yb = _edge_tables(xn_r8, kA[i], kB[i])
        s_r8 = _sc_gather_add(ya.reshape(N, 16), yb.reshape(N, 16),
                              edge_index).reshape(E // 8, 128)
        h, st = _stats_matmul([s_r8, xe_r8], [None, kC[i]], rows_per_blk=8000)
        xe_r8 = _apply_matmul(h, st, k2e[i], float(16 * E), rows_per_blk=8000,
                              resid=xe_r8, hscale=H)
        P = _sc_scatter(xe_r8.reshape(E, 16),
                        edge_index.reshape(2, E // 128, 128), zeros_rows)
        Pr = P.reshape(4, N // 8, 128)
        si = Pr[0] + Pr[2]
        sj = Pr[1] + Pr[3]
        xn_r8 = _node_double_layer([si, sj, xn_r8], [kU[i], kV[i], kR[i]],
                                   k2n[i], float(16 * N), resid=xn_r8, hscale=H)

    a = _close_mlp1(xn_r8, kron_close, kron_lin1, b1t)
    out = _close_mlp2(a.reshape(N, 256), w2t, b2, rows_per_blk=1000)
    return (out, jax.nn.sigmoid(alpha))


# speedup vs baseline: 1.0520x; 1.0164x over previous
import functools

import jax
import jax.numpy as jnp
from jax import lax
from jax.experimental import pallas as pl
from jax.experimental.pallas import tpu as pltpu
from jax.experimental.pallas import tpu_sc as plsc

N = 10000
E = 640000
H = 0.1
_INTERPRET = False


def _stats_matmul(xs, krons, rows_per_blk, offsets=None, rows=None):
    R = rows if rows is not None else xs[0].shape[0]
    if offsets is None:
        offsets = [0] * len(xs)
    Lout = next(k.shape[1] for k in krons if k is not None)
    nb = R // rows_per_blk
    assert R % rows_per_blk == 0

    real_krons = [k for k in krons if k is not None]

    def body(*refs):
        bi = pl.program_id(0)
        nx = len(xs)
        x_refs = refs[:nx]
        k_refs = list(refs[nx:nx + len(real_krons)])
        h_ref, st_ref = refs[nx + len(real_krons)], refs[nx + len(real_krons) + 1]
        h = jnp.zeros((rows_per_blk, Lout), jnp.float32)
        ki = 0
        for xr, kr in zip(x_refs, krons):
            if kr is None:
                h = h + xr[...]
            else:
                h = h + jnp.dot(xr[...], k_refs[ki][...],
                                preferred_element_type=jnp.float32)
                ki += 1
        h_ref[...] = h
        ps = jnp.sum(h, axis=0, keepdims=True)
        ps2 = jnp.sum(h * h, axis=0, keepdims=True)
        if Lout > 128:
            ps = ps.reshape(Lout // 128, 128).sum(axis=0, keepdims=True)
            ps2 = ps2.reshape(Lout // 128, 128).sum(axis=0, keepdims=True)

        @pl.when(bi == 0)
        def _():
            st_ref[...] = jnp.zeros((2, 128), jnp.float32)

        st_ref[0:1, :] += ps
        st_ref[1:2, :] += ps2

    in_specs = (
        [pl.BlockSpec((rows_per_blk, 128), functools.partial(lambda o, b: (b + o, 0), o))
         for o in offsets]
        + [pl.BlockSpec((128, Lout), lambda b: (0, 0)) for _ in real_krons]
    )
    out_specs = [
        pl.BlockSpec((rows_per_blk, Lout), lambda b: (b, 0)),
        pl.BlockSpec((2, 128), lambda b: (0, 0)),
    ]
    h, st = pl.pallas_call(
        body,
        grid=(nb,),
        in_specs=in_specs,
        out_specs=out_specs,
        out_shape=[
            jax.ShapeDtypeStruct((R, Lout), jnp.float32),
            jax.ShapeDtypeStruct((2, 128), jnp.float32),
        ],
        interpret=_INTERPRET,
    )(*xs, *real_krons)
    return h, st


def _apply_matmul(h, stats, kron2, count, rows_per_blk, resid=None, hscale=None):
    R, Lin = h.shape
    Lout = kron2.shape[1]
    nb = R // rows_per_blk
    assert R % rows_per_blk == 0

    def body(*refs):
        if resid is not None:
            h_ref, st_ref, k_ref, r_ref, o_ref = refs
        else:
            h_ref, st_ref, k_ref, o_ref = refs
            r_ref = None
        s = jnp.sum(st_ref[0, :])
        s2 = jnp.sum(st_ref[1, :])
        mean = s / count
        var = s2 / count - mean * mean
        inv = lax.rsqrt(var + 1e-5)
        g = jnp.tanh((h_ref[...] - mean) * inv)
        d = jnp.dot(g, k_ref[...], preferred_element_type=jnp.float32)
        if r_ref is not None:
            o_ref[...] = r_ref[...] + hscale * d
        else:
            o_ref[...] = d

    ins = [h, stats, kron2] + ([resid] if resid is not None else [])
    in_specs = [
        pl.BlockSpec((rows_per_blk, Lin), lambda b: (b, 0)),
        pl.BlockSpec((2, 128), lambda b: (0, 0)),
        pl.BlockSpec((Lin, Lout), lambda b: (0, 0)),
    ] + ([pl.BlockSpec((rows_per_blk, Lout), lambda b: (b, 0))] if resid is not None else [])
    out = pl.pallas_call(
        body,
        grid=(nb,),
        in_specs=in_specs,
        out_specs=pl.BlockSpec((rows_per_blk, Lout), lambda b: (b, 0)),
        out_shape=jax.ShapeDtypeStruct((R, Lout), jnp.float32),
        interpret=_INTERPRET,
    )(*ins)
    return out


def _open_stats(x_b3m, w1, blk_m):
    M = x_b3m.shape[2]
    nb = M // blk_m
    assert M % blk_m == 0

    def body(x_ref, w_ref, h_ref, st_ref, acc_ref):
        bi = pl.program_id(0)
        h = lax.dot_general(w_ref[...], x_ref[0],
                            (((1,), (0,)), ((), ())),
                            preferred_element_type=jnp.float32)
        h_ref[...] = h

        @pl.when(bi == 0)
        def _():
            acc_ref[0] = 0.0
            acc_ref[1] = 0.0

        acc_ref[0] += jnp.sum(h)
        acc_ref[1] += jnp.sum(h * h)

        @pl.when(bi == nb - 1)
        def _():
            o = jnp.ones((1, 128), jnp.float32)
            st_ref[0:1, :] = o * (acc_ref[0] / 128.0)
            st_ref[1:2, :] = o * (acc_ref[1] / 128.0)

    h, st = pl.pallas_call(
        body,
        grid=(nb,),
        in_specs=[
            pl.BlockSpec((1, 3, blk_m), lambda b: (0, 0, b)),
            pl.BlockSpec((16, 3), lambda b: (0, 0)),
        ],
        out_specs=[
            pl.BlockSpec((16, blk_m), lambda b: (0, b)),
            pl.BlockSpec((2, 128), lambda b: (0, 0)),
        ],
        out_shape=[
            jax.ShapeDtypeStruct((16, M), jnp.float32),
            jax.ShapeDtypeStruct((2, 128), jnp.float32),
        ],
        scratch_shapes=[pltpu.SMEM((2,), jnp.float32)],
        interpret=_INTERPRET,
    )(x_b3m, w1)
    return h, st


def _open_apply(h_cm, stats, w2, count, blk_m):
    M = h_cm.shape[1]
    nb = M // blk_m

    def body(h_ref, st_ref, w_ref, o_ref):
        s = jnp.sum(st_ref[0, :])
        s2 = jnp.sum(st_ref[1, :])
        mean = s / count
        inv = lax.rsqrt(s2 / count - mean * mean + 1e-5)
        g = jnp.tanh((h_ref[...] - mean) * inv)
        o_ref[...] = jnp.transpose(
            jnp.dot(w_ref[...], g, preferred_element_type=jnp.float32))

    return pl.pallas_call(
        body,
        grid=(nb,),
        in_specs=[
            pl.BlockSpec((16, blk_m), lambda b: (0, b)),
            pl.BlockSpec((2, 128), lambda b: (0, 0)),
            pl.BlockSpec((16, 16), lambda b: (0, 0)),
        ],
        out_specs=pl.BlockSpec((blk_m, 16), lambda b: (b, 0)),
        out_shape=jax.ShapeDtypeStruct((M, 16), jnp.float32),
        interpret=_INTERPRET,
    )(h_cm, stats, w2)


def _node_double_layer(xs, krons, kron2, count, resid=None, hscale=None,
                       next_kab=None):
    Lout = kron2.shape[1]
    R = xs[0].shape[0]
    n_tab = 2 if next_kab is not None else 0

    def body(*refs):
        nx = len(xs)
        x_refs = refs[:nx]
        k_refs = refs[nx:2 * nx]
        k2_ref = refs[2 * nx]
        pos = 2 * nx + 1
        r_ref = None
        if resid is not None:
            r_ref = refs[pos]
            pos += 1
        tab_refs = refs[pos:pos + n_tab]
        pos += n_tab
        o_ref = refs[pos]
        ya_ref = refs[pos + 1] if n_tab else None
        yb_ref = refs[pos + 2] if n_tab else None
        h = jnp.zeros((R, krons[0].shape[1]), jnp.float32)
        for xr, kr in zip(x_refs, k_refs):
            h = h + jnp.dot(xr[...], kr[...], preferred_element_type=jnp.float32)
        mean = jnp.sum(h) / count
        var = jnp.sum(h * h) / count - mean * mean
        g = jnp.tanh((h - mean) * lax.rsqrt(var + 1e-5))
        d = jnp.dot(g, k2_ref[...], preferred_element_type=jnp.float32)
        if r_ref is not None:
            xn_new = r_ref[...] + hscale * d
        else:
            xn_new = d
        o_ref[...] = xn_new
        if n_tab:
            ya_ref[...] = jnp.dot(xn_new, tab_refs[0][...],
                                  preferred_element_type=jnp.float32)
            yb_ref[...] = jnp.dot(xn_new, tab_refs[1][...],
                                  preferred_element_type=jnp.float32)

    ins = (list(xs) + list(krons) + [kron2]
           + ([resid] if resid is not None else [])
           + (list(next_kab) if next_kab is not None else []))
    sh = jax.ShapeDtypeStruct((R, Lout), jnp.float32)
    out_shape = [sh] * (1 + n_tab) if n_tab else sh
    out = pl.pallas_call(
        body,
        out_shape=out_shape,
        interpret=_INTERPRET,
    )(*ins)
    return out


def _edge_layer_fused(s_r8, xe_r8, kc, k2, count, rows_per_blk):
    R = s_r8.shape[0]
    nb = R // rows_per_blk
    assert R % rows_per_blk == 0

    def body(s_ref, xe_ref, kc_ref, k2_ref, o_ref, st):
        p = pl.program_id(0)
        b = pl.program_id(1)

        @pl.when((p == 0) & (b == 0))
        def _():
            st[0] = 0.0
            st[1] = 0.0

        @pl.when(p == 0)
        def _():
            h = s_ref[...] + jnp.dot(xe_ref[...], kc_ref[...],
                                     preferred_element_type=jnp.float32)
            o_ref[...] = h
            st[0] += jnp.sum(h)
            st[1] += jnp.sum(h * h)

        @pl.when(p == 1)
        def _():
            @pl.when(b == 0)
            def _():
                mean = st[0] / count
                st[2] = mean
                st[3] = lax.rsqrt(st[1] / count - mean * mean + 1e-5)

            g = jnp.tanh((s_ref[...] - st[2]) * st[3])
            o_ref[...] = xe_ref[...] + H * jnp.dot(
                g, k2_ref[...], preferred_element_type=jnp.float32)

    return pl.pallas_call(
        body,
        grid=(2, nb),
        in_specs=[
            pl.BlockSpec((rows_per_blk, 128), lambda p, b: (b, 0)),
            pl.BlockSpec((rows_per_blk, 128), lambda p, b: (b, 0)),
            pl.BlockSpec((128, 128), lambda p, b: (0, 0)),
            pl.BlockSpec((128, 128), lambda p, b: (0, 0)),
        ],
        out_specs=pl.BlockSpec((rows_per_blk, 128), lambda p, b: (b, 0)),
        out_shape=jax.ShapeDtypeStruct((R, 128), jnp.float32),
        scratch_shapes=[pltpu.SMEM((4,), jnp.float32)],
        input_output_aliases={0: 0},
        interpret=_INTERPRET,
    )(s_r8, xe_r8, kc, k2)


def _edge_tables(xn_r8, ka, kb):
    def body(x_ref, ka_ref, kb_ref, a_ref, b_ref):
        a_ref[...] = jnp.dot(x_ref[...], ka_ref[...],
                             preferred_element_type=jnp.float32)
        b_ref[...] = jnp.dot(x_ref[...], kb_ref[...],
                             preferred_element_type=jnp.float32)

    sh = jax.ShapeDtypeStruct(xn_r8.shape, jnp.float32)
    return pl.pallas_call(
        body, out_shape=[sh, sh], interpret=_INTERPRET,
    )(xn_r8, ka, kb)


def _close_mlp1(xn_r8, kron_close, kron_lin1, b1t):
    def body(x_ref, kc_ref, k1_ref, b_ref, o_ref):
        y = jnp.dot(x_ref[...], kc_ref[...], preferred_element_type=jnp.float32)
        t = jnp.dot(y, k1_ref[...], preferred_element_type=jnp.float32) + b_ref[...]
        o_ref[...] = jnp.where(t > 0, t, jnp.exp(jnp.minimum(t, 0.0)) - 1.0)

    return pl.pallas_call(
        body,
        out_shape=jax.ShapeDtypeStruct((xn_r8.shape[0], kron_lin1.shape[1]), jnp.float32),
        interpret=_INTERPRET,
    )(xn_r8, kron_close, kron_lin1, b1t)


def _close_mlp2(a, w2t, b2, rows_per_blk):
    R = a.shape[0]
    nb = R // rows_per_blk

    def body(a_ref, w_ref, b_ref, o_ref):
        z = jnp.dot(a_ref[...], w_ref[...], preferred_element_type=jnp.float32) + b_ref[...]
        m = jnp.max(z, axis=1, keepdims=True)
        lse = m + jnp.log(jnp.sum(jnp.exp(z - m), axis=1, keepdims=True))
        o_ref[...] = z - lse

    return pl.pallas_call(
        body,
        grid=(nb,),
        in_specs=[
            pl.BlockSpec((rows_per_blk, 256), lambda b: (b, 0)),
            pl.BlockSpec((256, 1024), lambda b: (0, 0)),
            pl.BlockSpec((1, 1024), lambda b: (0, 0)),
        ],
        out_specs=pl.BlockSpec((rows_per_blk, 1024), lambda b: (b, 0)),
        out_shape=jax.ShapeDtypeStruct((R, 1024), jnp.float32),
        interpret=_INTERPRET,
    )(a, w2t, b2)


_GATHER_WIN = 128
_SC_PARAMS = pltpu.CompilerParams(use_tc_tiling_on_sc=False)


_GGRP = 1280


def _sc_gather_add(tab_a, tab_b, edge_index):
    n_grp = E // _GGRP
    assert E % _GGRP == 0
    mesh = plsc.VectorSubcoreMesh(core_axis_name="c", subcore_axis_name="s")

    @functools.partial(
        pl.kernel,
        out_type=jax.ShapeDtypeStruct((E, 16), jnp.float32),
        mesh=mesh,
        scratch_types=[
            pltpu.VMEM((_GGRP,), jnp.int32),
            pltpu.VMEM((_GGRP,), jnp.int32),
            pltpu.VMEM((_GGRP, 16), jnp.float32),
            pltpu.SemaphoreType.DMA,
        ],
        compiler_params=_SC_PARAMS,
    )
    def k(a_hbm, b_hbm, ij_hbm, o_hbm, ivm, jvm, rows, sem):
        wid = lax.axis_index("c") * 16 + lax.axis_index("s")
        nk = _GGRP // 128

        @pl.loop(0, n_grp)
        def _(g):
            @pl.when(g % 32 == wid)
            def _():
                base = g * _GGRP
                pltpu.sync_copy(ij_hbm.at[0, pl.ds(base, _GGRP)], ivm)
                pltpu.sync_copy(ij_hbm.at[1, pl.ds(base, _GGRP)], jvm)
                hs = [
                    pltpu.async_copy(
                        a_hbm.at[ivm.at[pl.ds(k * 128, 128)]],
                        rows.at[pl.ds(k * 128, 128)], sem)
                    for k in range(nk)
                ]
                for h in hs:
                    h.wait()
                hs = [
                    pltpu.async_copy(
                        b_hbm.at[jvm.at[pl.ds(k * 128, 128)]],
                        rows.at[pl.ds(k * 128, 128)], sem, add=True)
                    for k in range(nk)
                ]
                for h in hs:
                    h.wait()
                pltpu.sync_copy(rows, o_hbm.at[pl.ds(base, _GGRP)])

    return k(tab_a, tab_b, edge_index)


_SGRP = 1024


def _sc_scatter(xe_rows, edge_index3, zeros_rows):
    mesh = plsc.VectorSubcoreMesh(core_axis_name="c", subcore_axis_name="s")
    n_grp = E // _SGRP
    kk = _SGRP // 128

    @functools.partial(
        pl.kernel,
        out_type=jax.ShapeDtypeStruct((2, 2, N, 16), jnp.float32),
        mesh=mesh,
        scratch_types=[
            pltpu.VMEM_SHARED((N, 16), jnp.float32),
            pltpu.VMEM_SHARED((N, 16), jnp.float32),
            pltpu.VMEM((kk, 128), jnp.int32),
            pltpu.VMEM((kk, 128), jnp.int32),
            pltpu.VMEM((_SGRP, 16), jnp.float32),
            pltpu.SemaphoreType.DMA,
            pltpu.SemaphoreType.DMA,
        ],
        compiler_params=_SC_PARAMS,
    )
    def k(xe_hbm, ij_hbm, z_hbm, o_hbm, acc_i, acc_j, ii, jj, rows, sem_l, sem_s):
        c = lax.axis_index("c")
        s = lax.axis_index("s")
        wid = c * 16 + s

        @pl.when(s == 0)
        def _():
            pltpu.sync_copy(z_hbm, acc_i)
            pltpu.sync_copy(z_hbm, acc_j)

        plsc.subcore_barrier()

        @pl.loop(0, n_grp)
        def _(g):
            @pl.when(g % 32 == wid)
            def _():
                h0 = pltpu.async_copy(xe_hbm.at[pl.ds(g * _SGRP, _SGRP)], rows,
                                      sem_l)
                pltpu.sync_copy(ij_hbm.at[0, pl.ds(g * kk, kk)], ii)
                pltpu.sync_copy(ij_hbm.at[1, pl.ds(g * kk, kk)], jj)
                h0.wait()
                hs = [
                    pltpu.async_copy(rows.at[pl.ds(k * 128, 128)],
                                     acc_i.at[ii.at[k]], sem_s, add=True)
                    for k in range(kk)
                ] + [
                    pltpu.async_copy(rows.at[pl.ds(k * 128, 128)],
                                     acc_j.at[jj.at[k]], sem_s, add=True)
                    for k in range(kk)
                ]
                for h in hs:
                    h.wait()

        plsc.subcore_barrier()

        @pl.when(s == 0)
        def _():
            pltpu.sync_copy(acc_i, o_hbm.at[c, 0])
            pltpu.sync_copy(acc_j, o_hbm.at[c, 1])

    return k(xe_rows, edge_index3, zeros_rows)


def _kron8(w):
    return jnp.kron(jnp.eye(8, dtype=jnp.float32), w.T)


def kernel(xn, xe, edge_index, K1Nopen, K2Nopen, K1Eopen, K2Eopen, KNclose,
           alpha, KE1, KE2, KN1, KN2, lin1_w, lin1_b, lin2_w, lin2_b):
    f32 = jnp.float32

    kA, kB, kC, k2e = [], [], [], []
    kU, kV, kR, k2n = [], [], [], []
    for i in range(KE1.shape[0]):
        P, C, G = KE1[i][:, 0:16], KE1[i][:, 16:32], KE1[i][:, 32:48]
        kA.append(_kron8(P / 2 + G))
        kB.append(_kron8(P / 2 - G))
        kC.append(_kron8(C))
        k2e.append(_kron8(KE2[i]))
        Pn, Qn, Rn = KN1[i][:, 0:16], KN1[i][:, 16:32], KN1[i][:, 32:48]
        kU.append(_kron8(Pn / 2 + Qn))
        kV.append(_kron8(Pn / 2 - Qn))
        kR.append(_kron8(Rn))
        k2n.append(_kron8(KN2[i]))
    kron_close = _kron8(KNclose)
    kron_lin1 = jnp.kron(jnp.eye(8, dtype=f32), lin1_w.T)
    b1t = jnp.tile(lin1_b, 8).reshape(1, 2048)
    w2t = lin2_w.T
    b2 = lin2_b.reshape(1, 1024)
    zeros_rows = jnp.zeros((N, 16), f32)

    hn, stn = _open_stats(xn, K1Nopen, blk_m=N)
    xn_r8 = _open_apply(hn, stn, K2Nopen, float(16 * N), blk_m=N).reshape(N // 8, 128)

    he, ste = _open_stats(xe, K1Eopen, blk_m=6400)
    xe_r8 = _open_apply(he, ste, K2Eopen, float(16 * E), blk_m=6400).reshape(E // 8, 128)

    nlayer = KE1.shape[0]
    ya, yb = _edge_tables(xn_r8, kA[0], kB[0])
    for i in range(nlayer):
        s_r8 = _sc_gather_add(ya.reshape(N, 16), yb.reshape(N, 16),
                              edge_index).reshape(E // 8, 128)
        xe_r8 = _edge_layer_fused(s_r8, xe_r8, kC[i], k2e[i], float(16 * E),
                                  rows_per_blk=8000)
        P = _sc_scatter(xe_r8.reshape(E, 16),
                        edge_index.reshape(2, E // 128, 128), zeros_rows)
        Pr = P.reshape(4, N // 8, 128)
        si = Pr[0] + Pr[2]
        sj = Pr[1] + Pr[3]
        nxt = (kA[i + 1], kB[i + 1]) if i + 1 < nlayer else None
        res = _node_double_layer([si, sj, xn_r8], [kU[i], kV[i], kR[i]],
                                 k2n[i], float(16 * N), resid=xn_r8, hscale=H,
                                 next_kab=nxt)
        if nxt is not None:
            xn_r8, ya, yb = res
        else:
            xn_r8 = res

    a = _close_mlp1(xn_r8, kron_close, kron_lin1, b1t)
    out = _close_mlp2(a.reshape(N, 256), w2t, b2, rows_per_blk=1000)
    return (out, jax.nn.sigmoid(alpha))
